# Initial kernel scaffold; baseline (speedup 1.0000x reference)
import jax
import jax.numpy as jnp
from jax.experimental import pallas as pl


def kernel(h, coords, edge_index, edge_attr, Win, bin_, Wout, bout, We1, be1, We2, be2, Wn1, bn1, Wn2, bn2, Wc1, bc1, Wc2):
    raise NotImplementedError("write your pallas kernel here")


# speedup vs baseline: 1.6945x; 1.6945x over previous
import functools

import jax
import jax.numpy as jnp
from jax import lax
from jax.experimental import pallas as pl
from jax.experimental.pallas import tpu as pltpu
from jax.experimental.pallas import tpu_sc as plsc

N = 10000
E = 320000
DIN = 128
HID = 128
DOUT = 128
ED = 4
L = 4

NPAD = 10240
TW = 256
NC = 2
NS = 16
NW = NC * NS
GCH = 128
KCH = 80
EPW = KCH * GCH
EPAD = NW * EPW
RPT = NPAD // NS

C_E = 1024
C_N = 1024

_mesh = plsc.VectorSubcoreMesh(core_axis_name="c", subcore_axis_name="s")
F32 = jnp.float32


def _silu(x):
    return x * jax.nn.sigmoid(x)


def _dot(a, b):
    return jnp.dot(a, b, preferred_element_type=F32)


@functools.partial(
    pl.kernel,
    mesh=_mesh,
    out_type=(
        jax.ShapeDtypeStruct((EPAD, TW), F32),
        jax.ShapeDtypeStruct((EPAD, TW), F32),
    ),
    scratch_types=[
        pltpu.VMEM((KCH, GCH), jnp.int32),
        pltpu.VMEM((KCH, GCH), jnp.int32),
        pltpu.VMEM((GCH, TW), F32),
        pltpu.VMEM((GCH, TW), F32),
        pltpu.SemaphoreType.DMA,
        pltpu.SemaphoreType.DMA,
    ],
)
def _sc_gather(t_hbm, col_hbm, row_hbm, outc_hbm, outr_hbm,
               idxc_v, idxr_v, bufc, bufr, semc, semr):
    wid = lax.axis_index("s") * NC + lax.axis_index("c")
    base = wid * EPW
    pltpu.sync_copy(col_hbm.at[pl.ds(wid * KCH, KCH)], idxc_v)
    pltpu.sync_copy(row_hbm.at[pl.ds(wid * KCH, KCH)], idxr_v)

    @pl.loop(0, KCH)
    def _(j):
        pltpu.async_copy(t_hbm.at[idxc_v.at[j]], bufc, semc).wait()
        pltpu.sync_copy(bufc, outc_hbm.at[pl.ds(base + j * GCH, GCH)])
        pltpu.async_copy(t_hbm.at[idxr_v.at[j]], bufr, semr).wait()
        pltpu.sync_copy(bufr, outr_hbm.at[pl.ds(base + j * GCH, GCH)])


@functools.partial(
    pl.kernel,
    mesh=_mesh,
    out_type=jax.ShapeDtypeStruct((NC, 2, NPAD, HID), F32),
    scratch_types=[
        pltpu.VMEM_SHARED((NPAD, HID), F32),
        pltpu.VMEM((KCH, GCH), jnp.int32),
        pltpu.VMEM((GCH, HID), F32),
    ],
)
def _sc_scatter(msg_m_hbm, msg_s_hbm, col_hbm, zeros_hbm, out_hbm,
                acc_sh, idx_v, buf):
    cid = lax.axis_index("c")
    sid = lax.axis_index("s")
    wid = sid * NC + cid
    base = wid * EPW
    pltpu.sync_copy(col_hbm.at[pl.ds(wid * KCH, KCH)], idx_v)

    def _phase(src_hbm, slot):
        pltpu.sync_copy(zeros_hbm.at[pl.ds(sid * RPT, RPT)],
                        acc_sh.at[pl.ds(sid * RPT, RPT)])
        plsc.subcore_barrier()

        @pl.loop(0, KCH)
        def _(j):
            pltpu.sync_copy(src_hbm.at[pl.ds(base + j * GCH, GCH)], buf)
            pltpu.sync_copy(buf, acc_sh.at[idx_v.at[j]], add=True)

        plsc.subcore_barrier()
        pltpu.sync_copy(acc_sh.at[pl.ds(sid * RPT, RPT)],
                        out_hbm.at[cid, slot, pl.ds(sid * RPT, RPT)])
        plsc.subcore_barrier()

    _phase(msg_m_hbm, 0)
    _phase(msg_s_hbm, 1)


def _embed_body(h_ref, c_ref, Win_ref, bin_ref, t_ref):
    h0 = _dot(h_ref[...], Win_ref[...]) + bin_ref[...]
    b = h0.shape[0]
    t_ref[...] = jnp.concatenate(
        [h0, c_ref[...], jnp.zeros((b, TW - 131), F32)], axis=1)


def _edge_body(tc_ref, tr_ref, ea_ref, Wa, Wb, wd, We, be1r, We2r, be2r,
               Wc1r, bc1r, Wc2r, m_ref, s_ref):
    tcv = tc_ref[...]
    trv = tr_ref[...]
    hc = tcv[:, :128]
    hr = trv[:, :128]
    cd = trv[:, 128:131] - tcv[:, 128:131]
    dist = jnp.sum(cd * cd, axis=1, keepdims=True)
    pre = (_dot(hc, Wa[...]) + _dot(hr, Wb[...]) + dist * wd[...]
           + _dot(ea_ref[...], We[...]) + be1r[...])
    m = _silu(pre)
    m = _silu(_dot(m, We2r[...]) + be2r[...])
    cw = _dot(_silu(_dot(m, Wc1r[...]) + bc1r[...]), Wc2r[...])
    cu = cd * cw
    b = m.shape[0]
    m_ref[...] = m
    s_ref[...] = jnp.concatenate(
        [cu, jnp.ones((b, 1), F32), jnp.zeros((b, HID - 4), F32)], axis=1)


def _node_body(t_ref, a0m_ref, a1m_ref, a0s_ref, a1s_ref, Wn1a, Wn1b, bn1r,
               Wn2r, bn2r, t_out_ref):
    tv = t_ref[...]
    h = tv[:, :128]
    coords = tv[:, 128:131]
    agg_feat = a0m_ref[...] + a1m_ref[...]
    s = a0s_ref[...] + a1s_ref[...]
    cnt = jnp.maximum(s[:, 3:4], 1.0)
    agg_coord = s[:, 0:3] / cnt
    u = _silu(_dot(h, Wn1a[...]) + _dot(agg_feat, Wn1b[...]) + bn1r[...])
    upd = _dot(u, Wn2r[...]) + bn2r[...]
    hn = h + upd
    cn = 2.0 * coords + agg_coord
    b = hn.shape[0]
    t_out_ref[...] = jnp.concatenate(
        [hn, cn, jnp.zeros((b, TW - 131), F32)], axis=1)


def _out_body(t_ref, Wout_ref, bout_ref, h_ref, c_ref):
    tv = t_ref[...]
    h_ref[...] = _dot(tv[:, :128], Wout_ref[...]) + bout_ref[...]
    c_ref[...] = tv[:, 128:131]


def _full(r, c):
    return pl.BlockSpec((r, c), lambda i: (0, 0))


def _tc_embed(h_pad, coords_pad, Win, bin_r):
    grid = (NPAD // C_N,)
    return pl.pallas_call(
        _embed_body,
        grid=grid,
        in_specs=[
            pl.BlockSpec((C_N, DIN), lambda i: (i, 0)),
            pl.BlockSpec((C_N, 3), lambda i: (i, 0)),
            _full(DIN, HID),
            _full(1, HID),
        ],
        out_specs=pl.BlockSpec((C_N, TW), lambda i: (i, 0)),
        out_shape=jax.ShapeDtypeStruct((NPAD, TW), F32),
    )(h_pad, coords_pad, Win, bin_r)


def _tc_edge(tcg, trg, ea_pad, Wa, Wb, wd, We, be1r, We2r, be2r, Wc1r,
             bc1r, Wc2r):
    grid = (EPAD // C_E,)
    return pl.pallas_call(
        _edge_body,
        grid=grid,
        in_specs=[
            pl.BlockSpec((C_E, TW), lambda i: (i, 0)),
            pl.BlockSpec((C_E, TW), lambda i: (i, 0)),
            pl.BlockSpec((C_E, ED), lambda i: (i, 0)),
            _full(HID, HID),
            _full(HID, HID),
            _full(1, HID),
            _full(ED, HID),
            _full(1, HID),
            _full(HID, HID),
            _full(1, HID),
            _full(HID, HID),
            _full(1, HID),
            _full(HID, 1),
        ],
        out_specs=[
            pl.BlockSpec((C_E, HID), lambda i: (i, 0)),
            pl.BlockSpec((C_E, HID), lambda i: (i, 0)),
        ],
        out_shape=[
            jax.ShapeDtypeStruct((EPAD, HID), F32),
            jax.ShapeDtypeStruct((EPAD, HID), F32),
        ],
    )(tcg, trg, ea_pad, Wa, Wb, wd, We, be1r, We2r, be2r, Wc1r, bc1r, Wc2r)


def _tc_node(t, a0m, a1m, a0s, a1s, Wn1a, Wn1b, bn1r, Wn2r, bn2r):
    grid = (NPAD // C_N,)
    return pl.pallas_call(
        _node_body,
        grid=grid,
        in_specs=[
            pl.BlockSpec((C_N, TW), lambda i: (i, 0)),
            pl.BlockSpec((C_N, HID), lambda i: (i, 0)),
            pl.BlockSpec((C_N, HID), lambda i: (i, 0)),
            pl.BlockSpec((C_N, HID), lambda i: (i, 0)),
            pl.BlockSpec((C_N, HID), lambda i: (i, 0)),
            _full(HID, HID),
            _full(HID, HID),
            _full(1, HID),
            _full(HID, HID),
            _full(1, HID),
        ],
        out_specs=pl.BlockSpec((C_N, TW), lambda i: (i, 0)),
        out_shape=jax.ShapeDtypeStruct((NPAD, TW), F32),
    )(t, a0m, a1m, a0s, a1s, Wn1a, Wn1b, bn1r, Wn2r, bn2r)


def _tc_out(t, Wout, bout_r):
    grid = (NPAD // C_N,)
    return pl.pallas_call(
        _out_body,
        grid=grid,
        in_specs=[
            pl.BlockSpec((C_N, TW), lambda i: (i, 0)),
            _full(HID, DOUT),
            _full(1, DOUT),
        ],
        out_specs=[
            pl.BlockSpec((C_N, DOUT), lambda i: (i, 0)),
            pl.BlockSpec((C_N, 3), lambda i: (i, 0)),
        ],
        out_shape=[
            jax.ShapeDtypeStruct((NPAD, DOUT), F32),
            jax.ShapeDtypeStruct((NPAD, 3), F32),
        ],
    )(t, Wout, bout_r)


def kernel(h, coords, edge_index, edge_attr, Win, bin_, Wout, bout,
           We1, be1, We2, be2, Wn1, bn1, Wn2, bn2, Wc1, bc1, Wc2):
    row = edge_index[0].astype(jnp.int32)
    col = edge_index[1].astype(jnp.int32)
    pad_e = EPAD - E
    col_pad = jnp.concatenate(
        [col, jnp.full((pad_e,), NPAD - 1, jnp.int32)]).reshape(EPAD // GCH, GCH)
    row_pad = jnp.concatenate(
        [row, jnp.zeros((pad_e,), jnp.int32)]).reshape(EPAD // GCH, GCH)
    ea_pad = jnp.concatenate(
        [edge_attr, jnp.zeros((pad_e, ED), F32)], axis=0)
    h_pad = jnp.concatenate([h, jnp.zeros((NPAD - N, DIN), F32)], axis=0)
    coords_pad = jnp.concatenate(
        [coords, jnp.zeros((NPAD - N, 3), F32)], axis=0)

    zeros128 = jnp.zeros((NPAD, HID), F32)

    t = _tc_embed(h_pad, coords_pad, Win, bin_.reshape(1, HID))

    for l in range(L):
        Wa = We1[l, 0:HID]
        Wb = We1[l, HID:2 * HID]
        wd = We1[l, 2 * HID:2 * HID + 1]
        We = We1[l, 2 * HID + 1:]
        tcg, trg = _sc_gather(t, col_pad, row_pad)
        msg_m, msg_s = _tc_edge(tcg, trg, ea_pad, Wa, Wb, wd, We,
                                be1[l].reshape(1, HID), We2[l],
                                be2[l].reshape(1, HID), Wc1[l],
                                bc1[l].reshape(1, HID), Wc2[l])
        agg = _sc_scatter(msg_m, msg_s, col_pad, zeros128)
        t = _tc_node(t, agg[0, 0], agg[1, 0], agg[0, 1], agg[1, 1],
                     Wn1[l, :HID], Wn1[l, HID:], bn1[l].reshape(1, HID),
                     Wn2[l], bn2[l].reshape(1, HID))

    h_out, c_out = _tc_out(t, Wout, bout.reshape(1, DOUT))
    return (h_out[:N], c_out[:N])


# speedup vs baseline: 7.2351x; 4.2699x over previous
import functools

import jax
import jax.numpy as jnp
from jax import lax
from jax.experimental import pallas as pl
from jax.experimental.pallas import tpu as pltpu
from jax.experimental.pallas import tpu_sc as plsc

N = 10000
E = 320000
DIN = 128
HID = 128
DOUT = 128
ED = 4
L = 4

NPAD = 10240
TW = 256
NC = 2
NS = 16
NW = NC * NS
GCH = 96
EPAD = 322560
NCHUNK = EPAD // GCH
RPT = NPAD // NS

C_E = 1024
C_N = 1024

_mesh = plsc.VectorSubcoreMesh(core_axis_name="c", subcore_axis_name="s")
F32 = jnp.float32


def _silu(x):
    return x * jax.nn.sigmoid(x)


def _dot(a, b):
    return jnp.dot(a, b, preferred_element_type=F32)


@functools.partial(
    pl.kernel,
    mesh=_mesh,
    out_type=(
        jax.ShapeDtypeStruct((EPAD, TW), F32),
        jax.ShapeDtypeStruct((EPAD, TW), F32),
    ),
    scratch_types=[
        pltpu.SemaphoreType.DMA,
        pltpu.SemaphoreType.DMA,
    ],
)
def _sc_gather(t_hbm, col_hbm, row_hbm, outc_hbm, outr_hbm, semc, semr):
    def body(ic_vmem, ir_vmem, oc_vmem, or_vmem):
        cpc = pltpu.async_copy(t_hbm.at[ic_vmem.at[0, 0]], oc_vmem, semc)
        cpr = pltpu.async_copy(t_hbm.at[ir_vmem.at[0, 0]], or_vmem, semr)
        cpc.wait()
        cpr.wait()

    pltpu.emit_pipeline(
        body,
        grid=(NCHUNK,),
        in_specs=[
            pl.BlockSpec((1, 1, GCH), lambda i: (i, 0, 0)),
            pl.BlockSpec((1, 1, GCH), lambda i: (i, 0, 0)),
        ],
        out_specs=[
            pl.BlockSpec((GCH, TW), lambda i: (i, 0)),
            pl.BlockSpec((GCH, TW), lambda i: (i, 0)),
        ],
        core_axis_name=("c", "s"),
        dimension_semantics=(pltpu.PARALLEL,),
    )(col_hbm, row_hbm, outc_hbm, outr_hbm)


@functools.partial(
    pl.kernel,
    mesh=_mesh,
    out_type=jax.ShapeDtypeStruct((NC, 2, NPAD, HID), F32),
    scratch_types=[
        pltpu.VMEM_SHARED((NPAD, HID), F32),
    ],
)
def _sc_scatter(msg_m_hbm, msg_s_hbm, col_hbm, zeros_hbm, out_hbm, acc_sh):
    cid = lax.axis_index("c")
    sid = lax.axis_index("s")

    def _phase(src_hbm, slot):
        pltpu.sync_copy(zeros_hbm.at[pl.ds(sid * RPT, RPT)],
                        acc_sh.at[pl.ds(sid * RPT, RPT)])
        plsc.subcore_barrier()

        def body(m_vmem, i_vmem):
            pltpu.sync_copy(m_vmem, acc_sh.at[i_vmem.at[0, 0]], add=True)

        pltpu.emit_pipeline(
            body,
            grid=(NCHUNK,),
            in_specs=[
                pl.BlockSpec((GCH, HID), lambda i: (i, 0)),
                pl.BlockSpec((1, 1, GCH), lambda i: (i, 0, 0)),
            ],
            out_specs=[],
            core_axis_name=("c", "s"),
            dimension_semantics=(pltpu.PARALLEL,),
        )(src_hbm, col_hbm)

        plsc.subcore_barrier()
        pltpu.sync_copy(acc_sh.at[pl.ds(sid * RPT, RPT)],
                        out_hbm.at[cid, slot, pl.ds(sid * RPT, RPT)])
        plsc.subcore_barrier()

    _phase(msg_m_hbm, 0)
    _phase(msg_s_hbm, 1)


def _embed_body(h_ref, c_ref, Win_ref, bin_ref, t_ref):
    h0 = _dot(h_ref[...], Win_ref[...]) + bin_ref[...]
    b = h0.shape[0]
    t_ref[...] = jnp.concatenate(
        [h0, c_ref[...], jnp.zeros((b, TW - 131), F32)], axis=1)


def _edge_body(tc_ref, tr_ref, ea_ref, Wa, Wb, wd, We, be1r, We2r, be2r,
               Wc1r, bc1r, Wc2r, m_ref, s_ref):
    tcv = tc_ref[...]
    trv = tr_ref[...]
    hc = tcv[:, :128]
    hr = trv[:, :128]
    cd = trv[:, 128:131] - tcv[:, 128:131]
    dist = jnp.sum(cd * cd, axis=1, keepdims=True)
    pre = (_dot(hc, Wa[...]) + _dot(hr, Wb[...]) + dist * wd[...]
           + _dot(ea_ref[...], We[...]) + be1r[...])
    m = _silu(pre)
    m = _silu(_dot(m, We2r[...]) + be2r[...])
    cw = _dot(_silu(_dot(m, Wc1r[...]) + bc1r[...]), Wc2r[...])
    cu = cd * cw
    b = m.shape[0]
    m_ref[...] = m
    s_ref[...] = jnp.concatenate(
        [cu, jnp.ones((b, 1), F32), jnp.zeros((b, HID - 4), F32)], axis=1)


def _node_body(t_ref, a0m_ref, a1m_ref, a0s_ref, a1s_ref, Wn1a, Wn1b, bn1r,
               Wn2r, bn2r, t_out_ref):
    tv = t_ref[...]
    h = tv[:, :128]
    coords = tv[:, 128:131]
    agg_feat = a0m_ref[...] + a1m_ref[...]
    s = a0s_ref[...] + a1s_ref[...]
    cnt = jnp.maximum(s[:, 3:4], 1.0)
    agg_coord = s[:, 0:3] / cnt
    u = _silu(_dot(h, Wn1a[...]) + _dot(agg_feat, Wn1b[...]) + bn1r[...])
    upd = _dot(u, Wn2r[...]) + bn2r[...]
    hn = h + upd
    cn = 2.0 * coords + agg_coord
    b = hn.shape[0]
    t_out_ref[...] = jnp.concatenate(
        [hn, cn, jnp.zeros((b, TW - 131), F32)], axis=1)


def _out_body(t_ref, Wout_ref, bout_ref, h_ref, c_ref):
    tv = t_ref[...]
    h_ref[...] = _dot(tv[:, :128], Wout_ref[...]) + bout_ref[...]
    c_ref[...] = tv[:, 128:131]


def _full(r, c):
    return pl.BlockSpec((r, c), lambda i: (0, 0))


def _tc_embed(h_pad, coords_pad, Win, bin_r):
    grid = (NPAD // C_N,)
    return pl.pallas_call(
        _embed_body,
        grid=grid,
        in_specs=[
            pl.BlockSpec((C_N, DIN), lambda i: (i, 0)),
            pl.BlockSpec((C_N, 3), lambda i: (i, 0)),
            _full(DIN, HID),
            _full(1, HID),
        ],
        out_specs=pl.BlockSpec((C_N, TW), lambda i: (i, 0)),
        out_shape=jax.ShapeDtypeStruct((NPAD, TW), F32),
    )(h_pad, coords_pad, Win, bin_r)


def _tc_edge(tcg, trg, ea_pad, Wa, Wb, wd, We, be1r, We2r, be2r, Wc1r,
             bc1r, Wc2r):
    grid = (EPAD // C_E,)
    return pl.pallas_call(
        _edge_body,
        grid=grid,
        in_specs=[
            pl.BlockSpec((C_E, TW), lambda i: (i, 0)),
            pl.BlockSpec((C_E, TW), lambda i: (i, 0)),
            pl.BlockSpec((C_E, ED), lambda i: (i, 0)),
            _full(HID, HID),
            _full(HID, HID),
            _full(1, HID),
            _full(ED, HID),
            _full(1, HID),
            _full(HID, HID),
            _full(1, HID),
            _full(HID, HID),
            _full(1, HID),
            _full(HID, 1),
        ],
        out_specs=[
            pl.BlockSpec((C_E, HID), lambda i: (i, 0)),
            pl.BlockSpec((C_E, HID), lambda i: (i, 0)),
        ],
        out_shape=[
            jax.ShapeDtypeStruct((EPAD, HID), F32),
            jax.ShapeDtypeStruct((EPAD, HID), F32),
        ],
    )(tcg, trg, ea_pad, Wa, Wb, wd, We, be1r, We2r, be2r, Wc1r, bc1r, Wc2r)


def _tc_node(t, a0m, a1m, a0s, a1s, Wn1a, Wn1b, bn1r, Wn2r, bn2r):
    grid = (NPAD // C_N,)
    return pl.pallas_call(
        _node_body,
        grid=grid,
        in_specs=[
            pl.BlockSpec((C_N, TW), lambda i: (i, 0)),
            pl.BlockSpec((C_N, HID), lambda i: (i, 0)),
            pl.BlockSpec((C_N, HID), lambda i: (i, 0)),
            pl.BlockSpec((C_N, HID), lambda i: (i, 0)),
            pl.BlockSpec((C_N, HID), lambda i: (i, 0)),
            _full(HID, HID),
            _full(HID, HID),
            _full(1, HID),
            _full(HID, HID),
            _full(1, HID),
        ],
        out_specs=pl.BlockSpec((C_N, TW), lambda i: (i, 0)),
        out_shape=jax.ShapeDtypeStruct((NPAD, TW), F32),
    )(t, a0m, a1m, a0s, a1s, Wn1a, Wn1b, bn1r, Wn2r, bn2r)


def _tc_out(t, Wout, bout_r):
    grid = (NPAD // C_N,)
    return pl.pallas_call(
        _out_body,
        grid=grid,
        in_specs=[
            pl.BlockSpec((C_N, TW), lambda i: (i, 0)),
            _full(HID, DOUT),
            _full(1, DOUT),
        ],
        out_specs=[
            pl.BlockSpec((C_N, DOUT), lambda i: (i, 0)),
            pl.BlockSpec((C_N, 3), lambda i: (i, 0)),
        ],
        out_shape=[
            jax.ShapeDtypeStruct((NPAD, DOUT), F32),
            jax.ShapeDtypeStruct((NPAD, 3), F32),
        ],
    )(t, Wout, bout_r)


def kernel(h, coords, edge_index, edge_attr, Win, bin_, Wout, bout,
           We1, be1, We2, be2, Wn1, bn1, Wn2, bn2, Wc1, bc1, Wc2):
    row = edge_index[0].astype(jnp.int32)
    col = edge_index[1].astype(jnp.int32)
    pad_e = EPAD - E
    col_pad = jnp.concatenate(
        [col, jnp.full((pad_e,), NPAD - 1, jnp.int32)]).reshape(NCHUNK, 1, GCH)
    row_pad = jnp.concatenate(
        [row, jnp.zeros((pad_e,), jnp.int32)]).reshape(NCHUNK, 1, GCH)
    ea_pad = jnp.concatenate(
        [edge_attr, jnp.zeros((pad_e, ED), F32)], axis=0)
    h_pad = jnp.concatenate([h, jnp.zeros((NPAD - N, DIN), F32)], axis=0)
    coords_pad = jnp.concatenate(
        [coords, jnp.zeros((NPAD - N, 3), F32)], axis=0)

    zeros128 = jnp.zeros((NPAD, HID), F32)

    t = _tc_embed(h_pad, coords_pad, Win, bin_.reshape(1, HID))

    for l in range(L):
        Wa = We1[l, 0:HID]
        Wb = We1[l, HID:2 * HID]
        wd = We1[l, 2 * HID:2 * HID + 1]
        We = We1[l, 2 * HID + 1:]
        tcg, trg = _sc_gather(t, col_pad, row_pad)
        msg_m, msg_s = _tc_edge(tcg, trg, ea_pad, Wa, Wb, wd, We,
                                be1[l].reshape(1, HID), We2[l],
                                be2[l].reshape(1, HID), Wc1[l],
                                bc1[l].reshape(1, HID), Wc2[l])
        agg = _sc_scatter(msg_m, msg_s, col_pad, zeros128)
        t = _tc_node(t, agg[0, 0], agg[1, 0], agg[0, 1], agg[1, 1],
                     Wn1[l, :HID], Wn1[l, HID:], bn1[l].reshape(1, HID),
                     Wn2[l], bn2[l].reshape(1, HID))

    h_out, c_out = _tc_out(t, Wout, bout.reshape(1, DOUT))
    return (h_out[:N], c_out[:N])
